# baseline (speedup 1.0000x reference)
import functools

import jax
import jax.numpy as jnp
from jax import lax
from jax.experimental import pallas as pl
from jax.experimental.pallas import tpu as pltpu
from jax.experimental.pallas import tpu_sc as plsc

NUM_CLASS = 100000
CTX_DIM = 512
N_CLS_CTX = 4
B = 4096
N_TOK = 77
ROW = N_TOK * CTX_DIM
CROW = N_CLS_CTX * CTX_DIM

NUM_CORES = 2
NUM_SUBCORES = 16
NUM_WORKERS = NUM_CORES * NUM_SUBCORES
PER_W = B // NUM_WORKERS
CHUNK = 8
N_CHUNKS = PER_W // CHUNK

LEN_PREFIX = 5 * CTX_DIM
LEN_S1 = 2 * CTX_DIM
LEN_S2 = 3 * CTX_DIM
LEN_SUFFIX = 55 * CTX_DIM
OFF_PREFIX = 0
OFF_C1 = OFF_PREFIX + LEN_PREFIX
OFF_S1 = OFF_C1 + CROW
OFF_C2 = OFF_S1 + LEN_S1
OFF_S2 = OFF_C2 + CROW
OFF_C3 = OFF_S2 + LEN_S2
OFF_SUFFIX = OFF_C3 + CROW


@functools.partial(
    pl.kernel,
    out_type=jax.ShapeDtypeStruct((B * ROW,), jnp.float32),
    mesh=plsc.VectorSubcoreMesh(core_axis_name="c", subcore_axis_name="s"),
    scratch_types=[
        pltpu.VMEM((PER_W,), jnp.int32),
        pltpu.VMEM((LEN_PREFIX,), jnp.float32),
        pltpu.VMEM((LEN_S1,), jnp.float32),
        pltpu.VMEM((LEN_S2,), jnp.float32),
        pltpu.VMEM((LEN_SUFFIX,), jnp.float32),
        pltpu.VMEM((CHUNK, 1, CROW), jnp.float32),
        pltpu.VMEM((CHUNK, 1, CROW), jnp.float32),
        pltpu.VMEM((CHUNK, 1, CROW), jnp.float32),
        pltpu.SemaphoreType.DMA,
        pltpu.SemaphoreType.DMA,
    ],
)
def _prompt_concat_sc(label_hbm, t1_hbm, t2_hbm, t3_hbm, pre_hbm, s1_hbm,
                      s2_hbm, suf_hbm, out_hbm, idx_v, pre_v, s1_v, s2_v,
                      suf_v, r1, r2, r3, gsem, ssem):
    wid = lax.axis_index("s") * NUM_CORES + lax.axis_index("c")
    base = wid * PER_W

    pltpu.sync_copy(label_hbm.at[pl.ds(base, PER_W)], idx_v)
    pltpu.sync_copy(pre_hbm, pre_v)
    pltpu.sync_copy(s1_hbm, s1_v)
    pltpu.sync_copy(s2_hbm, s2_v)
    pltpu.sync_copy(suf_hbm, suf_v)

    def chunk_body(c, carry):
        off = c * CHUNK
        idx_c = idx_v.at[pl.ds(off, CHUNK)]
        g1 = pltpu.async_copy(t1_hbm.at[idx_c], r1, gsem)
        g2 = pltpu.async_copy(t2_hbm.at[idx_c], r2, gsem)
        g3 = pltpu.async_copy(t3_hbm.at[idx_c], r3, gsem)
        g1.wait()
        g2.wait()
        g3.wait()
        stores = []
        for j in range(CHUNK):
            ob = (base + off + j) * ROW
            stores.append(pltpu.async_copy(
                pre_v, out_hbm.at[pl.ds(ob + OFF_PREFIX, LEN_PREFIX)], ssem))
            stores.append(pltpu.async_copy(
                r1.at[j, 0], out_hbm.at[pl.ds(ob + OFF_C1, CROW)], ssem))
            stores.append(pltpu.async_copy(
                s1_v, out_hbm.at[pl.ds(ob + OFF_S1, LEN_S1)], ssem))
            stores.append(pltpu.async_copy(
                r2.at[j, 0], out_hbm.at[pl.ds(ob + OFF_C2, CROW)], ssem))
            stores.append(pltpu.async_copy(
                s2_v, out_hbm.at[pl.ds(ob + OFF_S2, LEN_S2)], ssem))
            stores.append(pltpu.async_copy(
                r3.at[j, 0], out_hbm.at[pl.ds(ob + OFF_C3, CROW)], ssem))
            stores.append(pltpu.async_copy(
                suf_v, out_hbm.at[pl.ds(ob + OFF_SUFFIX, LEN_SUFFIX)], ssem))
        for s in stores:
            s.wait()
        return carry

    lax.fori_loop(0, N_CHUNKS, chunk_body, 0)


def kernel(label, cls_ctx, cls_ctx2, cls_ctx3, token_prefix, token_suffix_1,
           token_suffix_2, token_suffix):
    out = _prompt_concat_sc(
        label.astype(jnp.int32),
        cls_ctx.reshape(NUM_CLASS, 1, CROW),
        cls_ctx2.reshape(NUM_CLASS, 1, CROW),
        cls_ctx3.reshape(NUM_CLASS, 1, CROW),
        token_prefix.reshape(LEN_PREFIX),
        token_suffix_1.reshape(LEN_S1),
        token_suffix_2.reshape(LEN_S2),
        token_suffix.reshape(LEN_SUFFIX),
    )
    return out.reshape(B, N_TOK, CTX_DIM)


# speedup vs baseline: 3.9401x; 3.9401x over previous
import functools

import jax
import jax.numpy as jnp
from jax.experimental import pallas as pl
from jax.experimental.pallas import tpu as pltpu

NUM_CLASS = 100000
CTX_DIM = 512
N_CLS_CTX = 4
B = 4096
N_TOK = 77


def _assemble(lbl_sref, c1, c2, c3, pre, s1, s2, suf, out):
    out[0, 0:5] = pre[0]
    out[0, 5:9] = c1[0]
    out[0, 9:11] = s1[0]
    out[0, 11:15] = c2[0]
    out[0, 15:18] = s2[0]
    out[0, 18:22] = c3[0]
    out[0, 22:77] = suf[0]


@jax.jit
def _prompt_concat(label, cls_ctx, cls_ctx2, cls_ctx3, token_prefix,
                   token_suffix_1, token_suffix_2, token_suffix):
    tbl_spec = pl.BlockSpec((1, N_CLS_CTX, CTX_DIM),
                            lambda i, lbl: (lbl[i], 0, 0))
    grid_spec = pltpu.PrefetchScalarGridSpec(
        num_scalar_prefetch=1,
        grid=(B,),
        in_specs=[
            tbl_spec,
            tbl_spec,
            tbl_spec,
            pl.BlockSpec((1, 5, CTX_DIM), lambda i, lbl: (0, 0, 0)),
            pl.BlockSpec((1, 2, CTX_DIM), lambda i, lbl: (0, 0, 0)),
            pl.BlockSpec((1, 3, CTX_DIM), lambda i, lbl: (0, 0, 0)),
            pl.BlockSpec((1, 55, CTX_DIM), lambda i, lbl: (0, 0, 0)),
        ],
        out_specs=pl.BlockSpec((1, N_TOK, CTX_DIM), lambda i, lbl: (i, 0, 0)),
    )
    return pl.pallas_call(
        _assemble,
        grid_spec=grid_spec,
        out_shape=jax.ShapeDtypeStruct((B, N_TOK, CTX_DIM), jnp.float32),
        compiler_params=pltpu.CompilerParams(
            dimension_semantics=("arbitrary",)),
    )(label.astype(jnp.int32), cls_ctx, cls_ctx2, cls_ctx3, token_prefix,
      token_suffix_1, token_suffix_2, token_suffix)


def kernel(label, cls_ctx, cls_ctx2, cls_ctx3, token_prefix, token_suffix_1,
           token_suffix_2, token_suffix):
    return _prompt_concat(label, cls_ctx, cls_ctx2, cls_ctx3, token_prefix,
                          token_suffix_1, token_suffix_2, token_suffix)


# speedup vs baseline: 29.0902x; 7.3831x over previous
import functools

import jax
import jax.numpy as jnp
from jax import lax
from jax.experimental import pallas as pl
from jax.experimental.pallas import tpu as pltpu
from jax.experimental.pallas import tpu_sc as plsc

NUM_CLASS = 100000
CTX_DIM = 512
N_CLS_CTX = 4
B = 4096
N_TOK = 77

NUM_CORES = 2
NUM_SUBCORES = 16
NUM_WORKERS = NUM_CORES * NUM_SUBCORES
PER_W = B // NUM_WORKERS
CHUNK = 16
N_CHUNKS = PER_W // CHUNK


@functools.partial(
    pl.kernel,
    out_type=[jax.ShapeDtypeStruct((B, N_CLS_CTX, CTX_DIM), jnp.float32)] * 3,
    mesh=plsc.VectorSubcoreMesh(core_axis_name="c", subcore_axis_name="s"),
    scratch_types=[
        pltpu.VMEM((PER_W,), jnp.int32),
        pltpu.VMEM((CHUNK, N_CLS_CTX, CTX_DIM), jnp.float32),
        pltpu.VMEM((CHUNK, N_CLS_CTX, CTX_DIM), jnp.float32),
        pltpu.VMEM((CHUNK, N_CLS_CTX, CTX_DIM), jnp.float32),
        pltpu.SemaphoreType.DMA,
        pltpu.SemaphoreType.DMA,
    ],
)
def _gather_sc(label_hbm, t1_hbm, t2_hbm, t3_hbm, c1_out, c2_out, c3_out,
               idx_v, r1, r2, r3, gsem, ssem):
    wid = lax.axis_index("s") * NUM_CORES + lax.axis_index("c")
    base = wid * PER_W
    pltpu.sync_copy(label_hbm.at[pl.ds(base, PER_W)], idx_v)

    def chunk_body(c, carry):
        off = c * CHUNK
        idx_c = idx_v.at[pl.ds(off, CHUNK)]
        g1 = pltpu.async_copy(t1_hbm.at[idx_c], r1, gsem)
        g2 = pltpu.async_copy(t2_hbm.at[idx_c], r2, gsem)
        g3 = pltpu.async_copy(t3_hbm.at[idx_c], r3, gsem)
        g1.wait()
        g2.wait()
        g3.wait()
        s1 = pltpu.async_copy(r1, c1_out.at[pl.ds(base + off, CHUNK)], ssem)
        s2 = pltpu.async_copy(r2, c2_out.at[pl.ds(base + off, CHUNK)], ssem)
        s3 = pltpu.async_copy(r3, c3_out.at[pl.ds(base + off, CHUNK)], ssem)
        s1.wait()
        s2.wait()
        s3.wait()
        return carry

    lax.fori_loop(0, N_CHUNKS, chunk_body, 0)


def kernel(label, cls_ctx, cls_ctx2, cls_ctx3, token_prefix, token_suffix_1,
           token_suffix_2, token_suffix):
    c1, c2, c3 = _gather_sc(label.astype(jnp.int32), cls_ctx, cls_ctx2,
                            cls_ctx3)
    prefix = jnp.broadcast_to(token_prefix, (B, 5, CTX_DIM))
    s1 = jnp.broadcast_to(token_suffix_1, (B, 2, CTX_DIM))
    s2 = jnp.broadcast_to(token_suffix_2, (B, 3, CTX_DIM))
    suffix = jnp.broadcast_to(token_suffix, (B, 55, CTX_DIM))
    return jnp.concatenate([prefix, c1, s1, c2, s2, c3, suffix], axis=1)
